# baseline (speedup 1.0000x reference)
import functools
import numpy as np
import jax
import jax.numpy as jnp
from jax import lax
from jax.experimental import pallas as pl
from jax.experimental.pallas import tpu as pltpu
from jax.experimental.pallas import tpu_sc as plsc

_DIM = 256
_G = 16
_DPG = _DIM // _G

_NC = 2
_NS = 16
_NW = _NC * _NS

_ROWS = 32 * 1024
_ROWS_PER_W = _ROWS // _NW
_T = 128
_CHUNKS = _ROWS_PER_W // _T


def _sc_body(x_hbm, q_hbm, idx_hbm, xin, qbuf, idxbuf):
    cid = lax.axis_index("c")
    sid = lax.axis_index("s")
    wid = sid * _NC + cid
    g_iota = lax.iota(jnp.int32, _G)
    col_base = g_iota * _DPG

    def chunk_body(cidx, carry):
        rbase = wid * _ROWS_PER_W + cidx * _T
        pltpu.sync_copy(x_hbm.at[pl.ds(rbase * _DIM, _T * _DIM)], xin)

        def tok_body(t, carry2):
            toff = t * _DIM
            acc = jnp.zeros((_G,), jnp.int32)
            for j in range(_DPG):
                ix = toff + col_base + j
                v = plsc.load_gather(xin, [ix])
                m = v > 0
                plsc.store_scatter(
                    qbuf, [ix],
                    jnp.where(m, jnp.float32(0.25), jnp.float32(-0.25)))
                acc = acc * 2 + m.astype(jnp.int32)
            t_splat = jnp.full((_G,), t, jnp.int32)
            plsc.store_scatter(idxbuf, [g_iota, t_splat], acc)
            return carry2

        lax.fori_loop(0, _T, tok_body, 0)
        pltpu.sync_copy(qbuf, q_hbm.at[pl.ds(rbase * _DIM, _T * _DIM)])
        pltpu.sync_copy(idxbuf, idx_hbm.at[:, pl.ds(rbase, _T)])
        return carry

    lax.fori_loop(0, _CHUNKS, chunk_body, 0)


@jax.jit
def _sc_call(xf):
    mesh = plsc.VectorSubcoreMesh(core_axis_name="c", subcore_axis_name="s")
    run = pl.kernel(
        _sc_body,
        out_type=[
            jax.ShapeDtypeStruct((_ROWS * _DIM,), jnp.float32),
            jax.ShapeDtypeStruct((_G, _ROWS), jnp.int32),
        ],
        mesh=mesh,
        scratch_types=[
            pltpu.VMEM((_T * _DIM,), jnp.float32),
            pltpu.VMEM((_T * _DIM,), jnp.float32),
            pltpu.VMEM((_G, _T), jnp.int32),
        ],
        compiler_params=pltpu.CompilerParams(needs_layout_passes=False),
    )
    return run(xf)


def kernel(x):
    b, n, dim = x.shape
    qf, idx = _sc_call(x.reshape(-1))
    quantized = qf.reshape(b, n, dim)
    all_indices = idx.reshape(_G, b, n)
    aux_losses = jnp.zeros((_G,), dtype=jnp.float32)
    return (quantized, all_indices, aux_losses)


# speedup vs baseline: 1.3790x; 1.3790x over previous
import functools
import numpy as np
import jax
import jax.numpy as jnp
from jax import lax
from jax.experimental import pallas as pl
from jax.experimental.pallas import tpu as pltpu
from jax.experimental.pallas import tpu_sc as plsc

_DIM = 256
_G = 16
_DPG = _DIM // _G

_NC = 2
_NS = 16
_NW = _NC * _NS

_ROWS = 32 * 1024
_ROWS_PER_W = _ROWS // _NW
_T = 128
_CHUNKS = _ROWS_PER_W // _T


def _sc_body(x_hbm, q_hbm, idx_hbm, xin, qbuf, idxbuf):
    cid = lax.axis_index("c")
    sid = lax.axis_index("s")
    wid = sid * _NC + cid
    g_iota = lax.iota(jnp.int32, _G)
    col_base = g_iota * _DPG

    def chunk_body(cidx, carry):
        rbase = wid * _ROWS_PER_W + cidx * _T
        pltpu.sync_copy(x_hbm.at[pl.ds(rbase * _DIM, _T * _DIM)], xin)

        @plsc.parallel_loop(0, _T, unroll=4)
        def tok_body(t):
            toff = t * _DIM
            terms = []
            for j in range(_DPG):
                ix = toff + col_base + j
                v = plsc.load_gather(xin, [ix])
                m = v > 0
                plsc.store_scatter(
                    qbuf, [ix],
                    jnp.where(m, jnp.float32(0.25), jnp.float32(-0.25)))
                terms.append(jnp.where(m, jnp.int32(1 << (_DPG - 1 - j)),
                                       jnp.int32(0)))
            while len(terms) > 1:
                terms = [terms[k] + terms[k + 1]
                         for k in range(0, len(terms), 2)]
            t_splat = jnp.full((_G,), t, jnp.int32)
            plsc.store_scatter(idxbuf, [g_iota, t_splat], terms[0])
        pltpu.sync_copy(qbuf, q_hbm.at[pl.ds(rbase * _DIM, _T * _DIM)])
        pltpu.sync_copy(idxbuf, idx_hbm.at[:, pl.ds(rbase, _T)])
        return carry

    lax.fori_loop(0, _CHUNKS, chunk_body, 0)


@jax.jit
def _sc_call(xf):
    mesh = plsc.VectorSubcoreMesh(core_axis_name="c", subcore_axis_name="s")
    run = pl.kernel(
        _sc_body,
        out_type=[
            jax.ShapeDtypeStruct((_ROWS * _DIM,), jnp.float32),
            jax.ShapeDtypeStruct((_G, _ROWS), jnp.int32),
        ],
        mesh=mesh,
        scratch_types=[
            pltpu.VMEM((_T * _DIM,), jnp.float32),
            pltpu.VMEM((_T * _DIM,), jnp.float32),
            pltpu.VMEM((_G, _T), jnp.int32),
        ],
        compiler_params=pltpu.CompilerParams(needs_layout_passes=False),
    )
    return run(xf)


def kernel(x):
    b, n, dim = x.shape
    qf, idx = _sc_call(x.reshape(-1))
    quantized = qf.reshape(b, n, dim)
    all_indices = idx.reshape(_G, b, n)
    aux_losses = jnp.zeros((_G,), dtype=jnp.float32)
    return (quantized, all_indices, aux_losses)
